# baseline (speedup 1.0000x reference)
import functools

import jax
import jax.numpy as jnp
from jax import lax
from jax.experimental import pallas as pl
from jax.experimental.pallas import tpu as pltpu

_A = 129
_AP = 136
_DIM = 256
_H = 4
_C = 64
_L = 2
_NEG = -1e9


def _ln(x, g, b, eps=1e-5):
    mu = jnp.mean(x, axis=-1, keepdims=True)
    var = jnp.mean((x - mu) ** 2, axis=-1, keepdims=True)
    return (x - mu) / jnp.sqrt(var + eps) * g + b


def _gnn_body(agents_ref, wn_ref, bn_ref, gn_ref, ben_ref,
              we_ref, beg_ref, ge_ref, bee_ref,
              wl_ref, asrc_ref, adst_ref, gbias_ref,
              wout_ref, bout_ref, out_ref):
    ag = agents_ref[0]
    agT = ag.T

    px_c = ag[:, 0:1]
    py_c = ag[:, 1:2]
    px_r = agT[0:1, :]
    py_r = agT[1:2, :]
    dx = px_c - px_r
    dy = py_c - py_r
    dist = jnp.sqrt(dx * dx + dy * dy + 1e-12)
    row_id = lax.broadcasted_iota(jnp.int32, (_AP, _AP), 0)
    col_id = lax.broadcasted_iota(jnp.int32, (_AP, _AP), 1)
    eye_m = row_id == col_id
    adj = ((dist < 50.0) & (~eye_m)) | eye_m
    mask = adj & (col_id < _A)

    hn = jnp.maximum(jnp.dot(ag, wn_ref[...],
                             preferred_element_type=jnp.float32)
                     + bn_ref[...], 0.0)
    hn = _ln(hn, gn_ref[...], ben_ref[...])
    he = jnp.maximum(jnp.dot(ag, we_ref[...],
                             preferred_element_type=jnp.float32)
                     + beg_ref[...], 0.0)
    he = _ln(he, ge_ref[...], bee_ref[...])
    is_ego = lax.broadcasted_iota(jnp.int32, (_AP, 1), 0) == 0
    h = jnp.where(is_ego, he, hn)

    for l in range(_L):
        x = jnp.dot(h, wl_ref[l], preferred_element_type=jnp.float32)
        a_dst = jnp.dot(x, adst_ref[l], preferred_element_type=jnp.float32)
        a_src = jnp.dot(x, asrc_ref[l], preferred_element_type=jnp.float32)
        a_srcT = a_src.T
        outs = []
        for hh in range(_H):
            lg = a_dst[:, hh:hh + 1] + a_srcT[hh:hh + 1, :]
            lg = jnp.where(lg >= 0.0, lg, 0.2 * lg)
            lg = jnp.where(mask, lg, _NEG)
            m = jnp.max(lg, axis=1, keepdims=True)
            e = jnp.exp(lg - m)
            s = jnp.sum(e, axis=1, keepdims=True)
            alpha = e / s
            outs.append(jnp.dot(alpha, x[:, hh * _C:(hh + 1) * _C],
                                preferred_element_type=jnp.float32))
        out = jnp.concatenate(outs, axis=1)
        h = jnp.maximum(out + gbias_ref[l:l + 1, :], 0.0)

    enc = jnp.dot(h, wout_ref[...], preferred_element_type=jnp.float32) \
        + bout_ref[...]
    out_ref[0] = enc


@jax.jit
def kernel(ego_agent_past, neighbor_agents_past, W_node, b_node, g_node,
           be_node, W_ego, b_ego, g_ego, be_ego, gat_W, gat_att_src,
           gat_att_dst, gat_bias, W_out, b_out):
    B = ego_agent_past.shape[0]
    ego_last = ego_agent_past[:, -1, :5]
    nb_last = neighbor_agents_past[:, :, -1, :5]
    agents = jnp.concatenate([ego_last[:, None, :], nb_last], axis=1)
    agents = jnp.pad(agents, ((0, 0), (0, _AP - _A), (0, 0)))

    Wn = W_node[:5]
    We = W_ego[:5]

    L, dim, H, C = gat_W.shape
    Wl = gat_W.reshape(L, dim, H * C)
    eyeH = jnp.eye(H, dtype=gat_W.dtype)
    Asrc = (gat_att_src[:, :, :, None] * eyeH[None, :, None, :]
            ).reshape(L, H * C, H)
    Adst = (gat_att_dst[:, :, :, None] * eyeH[None, :, None, :]
            ).reshape(L, H * C, H)

    row = lambda v: v.reshape(1, -1)
    const = lambda *dims: pl.BlockSpec(dims, lambda b: (0,) * len(dims))

    out = pl.pallas_call(
        _gnn_body,
        grid=(B,),
        in_specs=[
            pl.BlockSpec((1, _AP, 5), lambda b: (b, 0, 0)),
            const(5, dim), const(1, dim), const(1, dim), const(1, dim),
            const(5, dim), const(1, dim), const(1, dim), const(1, dim),
            const(L, dim, H * C), const(L, H * C, H), const(L, H * C, H),
            const(L, dim),
            const(dim, dim), const(1, dim),
        ],
        out_specs=pl.BlockSpec((1, _AP, dim), lambda b: (b, 0, 0)),
        out_shape=jax.ShapeDtypeStruct((B, _AP, dim), jnp.float32),
        compiler_params=pltpu.CompilerParams(
            dimension_semantics=("parallel",),
        ),
    )(agents, Wn, row(b_node), row(g_node), row(be_node),
      We, row(b_ego), row(g_ego), row(be_ego),
      Wl, Asrc, Adst, gat_bias,
      W_out, row(b_out))
    return out[:, :_A, :]


# speedup vs baseline: 1.5751x; 1.5751x over previous
import jax
import jax.numpy as jnp
from jax import lax
from jax.experimental import pallas as pl
from jax.experimental.pallas import tpu as pltpu

_B = 16
_A = 129
_AP = 136
_S4 = _AP * 4
_DIM = 256
_H = 4
_C = 64
_L = 2
_NEG = -1e9


def _ln(x, g, b, eps=1e-5):
    mu = jnp.mean(x, axis=-1, keepdims=True)
    var = jnp.mean((x - mu) ** 2, axis=-1, keepdims=True)
    return (x - mu) / jnp.sqrt(var + eps) * g + b


def _gnn_body(agents_ref, ego_ref, wn_ref, bn_ref, gn_ref, ben_ref,
              we_ref, beg_ref, ge_ref, bee_ref,
              wl_ref, asrc_ref, adst_ref, gbias_ref,
              wout_ref, bout_ref, out_ref, h_scr, madd_scr):
    f32 = jnp.float32

    row_id = lax.broadcasted_iota(jnp.int32, (_AP, _AP), 0)
    col_id = lax.broadcasted_iota(jnp.int32, (_AP, _AP), 1)
    eye_m = row_id == col_id
    col_ok = col_id < _A
    for b in range(_B):
        ag = agents_ref[b * _AP:(b + 1) * _AP, :]
        agT = ag.T
        dx = ag[:, 0:1] - agT[0:1, :]
        dy = ag[:, 1:2] - agT[1:2, :]
        dist = jnp.sqrt(dx * dx + dy * dy + 1e-12)
        mask = ((((dist < 50.0) & (~eye_m)) | eye_m) & col_ok)
        madd = jnp.where(mask, 0.0, _NEG).astype(f32)
        madd4 = jnp.concatenate([madd, madd, madd, madd], axis=0)
        madd_scr[b * _S4:(b + 1) * _S4, :] = madd4

    ag_all = agents_ref[...]
    hn = jnp.maximum(jnp.dot(ag_all, wn_ref[...],
                             preferred_element_type=f32) + bn_ref[...], 0.0)
    h_scr[...] = _ln(hn, gn_ref[...], ben_ref[...])
    he = jnp.maximum(jnp.dot(ego_ref[...], we_ref[...],
                             preferred_element_type=f32) + beg_ref[...], 0.0)
    he = _ln(he, ge_ref[...], bee_ref[...])
    for b in range(_B):
        h_scr[b * _AP:b * _AP + 1, :] = he[b:b + 1, :]

    for l in range(_L):
        h_all = h_scr[...]
        x_all = jnp.dot(h_all, wl_ref[l], preferred_element_type=f32)
        a_dst = jnp.dot(x_all, adst_ref[l], preferred_element_type=f32)
        a_src = jnp.dot(x_all, asrc_ref[l], preferred_element_type=f32)
        for b in range(_B):
            sl = slice(b * _AP, (b + 1) * _AP)
            xb = x_all[sl]
            ad = a_dst[sl]
            asT = a_src[sl].T
            lg = jnp.concatenate(
                [ad[:, h:h + 1] + asT[h:h + 1, :] for h in range(_H)],
                axis=0)
            lg = jnp.maximum(lg, 0.2 * lg) \
                + madd_scr[b * _S4:(b + 1) * _S4, :]
            m = jnp.max(lg, axis=1, keepdims=True)
            e = jnp.exp(lg - m)
            s = jnp.sum(e, axis=1, keepdims=True)
            big = jnp.dot(e, xb, preferred_element_type=f32)
            ob = jnp.concatenate(
                [big[h * _AP:(h + 1) * _AP, h * _C:(h + 1) * _C]
                 / s[h * _AP:(h + 1) * _AP, :] for h in range(_H)],
                axis=1)
            h_scr[sl, :] = jnp.maximum(ob + gbias_ref[l:l + 1, :], 0.0)

    out_ref[...] = jnp.dot(h_scr[...], wout_ref[...],
                           preferred_element_type=f32) + bout_ref[...]


@jax.jit
def kernel(ego_agent_past, neighbor_agents_past, W_node, b_node, g_node,
           be_node, W_ego, b_ego, g_ego, be_ego, gat_W, gat_att_src,
           gat_att_dst, gat_bias, W_out, b_out):
    ego_last = ego_agent_past[:, -1, :5]
    nb_last = neighbor_agents_past[:, :, -1, :5]
    agents = jnp.concatenate([ego_last[:, None, :], nb_last], axis=1)
    agents = jnp.pad(agents, ((0, 0), (0, _AP - _A), (0, 0)))
    agents = agents.reshape(_B * _AP, 5)

    Wn = W_node[:5]
    We = W_ego[:5]

    L, dim, H, C = gat_W.shape
    Wl = gat_W.reshape(L, dim, H * C)
    eyeH = jnp.eye(H, dtype=gat_W.dtype)
    Asrc = (gat_att_src[:, :, :, None] * eyeH[None, :, None, :]
            ).reshape(L, H * C, H)
    Adst = (gat_att_dst[:, :, :, None] * eyeH[None, :, None, :]
            ).reshape(L, H * C, H)

    row = lambda v: v.reshape(1, -1)

    out = pl.pallas_call(
        _gnn_body,
        out_shape=jax.ShapeDtypeStruct((_B * _AP, _DIM), jnp.float32),
        scratch_shapes=[
            pltpu.VMEM((_B * _AP, _DIM), jnp.float32),
            pltpu.VMEM((_B * _S4, _AP), jnp.float32),
        ],
    )(agents, ego_last, Wn, row(b_node), row(g_node), row(be_node),
      We, row(b_ego), row(g_ego), row(be_ego),
      Wl, Asrc, Adst, gat_bias,
      W_out, row(b_out))
    return out.reshape(_B, _AP, _DIM)[:, :_A, :]
